# baseline (speedup 1.0000x reference)
import math
import functools

import numpy as np
import jax
import jax.numpy as jnp
from jax.experimental import pallas as pl
from jax.experimental.pallas import tpu as pltpu

L = 2048
D = 1024
H = 16
DK = 64
U = max(1, int(5 * math.log(L + 1)))
SAMPLE_K = max(1, int(5 * math.log(L + 1)))
KB = 512
NKB = L // KB
LB = 256
UP = 64
SCALE = 1.0 / math.sqrt(DK)

def _threefry_pair(k0, k1, x0, x1):
    rotations = [(13, 15, 26, 6), (17, 29, 16, 24)]
    ks = [np.uint32(k0), np.uint32(k1),
          np.uint32(k0) ^ np.uint32(k1) ^ np.uint32(0x1BD11BDA)]
    x = [x0.astype(np.uint32).copy(), x1.astype(np.uint32).copy()]
    with np.errstate(over="ignore"):
        x[0] += ks[0]
        x[1] += ks[1]
        for i in range(5):
            for r in rotations[i % 2]:
                x[0] += x[1]
                x[1] = (x[1] << np.uint32(r)) | (x[1] >> np.uint32(32 - r))
                x[1] ^= x[0]
            x[0] += ks[(i + 1) % 3]
            x[1] += ks[(i + 2) % 3] + np.uint32(i + 1)
    return x[0], x[1]


_cntT_cache = None


def _cntT_host():
    global _cntT_cache
    if _cntT_cache is None:
        b0, b1 = _threefry_pair(0, 42, np.zeros(2, np.uint32),
                                np.arange(2, dtype=np.uint32))
        n = L * SAMPLE_K
        c0, c1 = _threefry_pair(b0[1], b1[1], np.zeros(n, np.uint32),
                                np.arange(n, dtype=np.uint32))
        idx = ((c0 ^ c1) % np.uint32(L)).astype(np.int32).reshape(L, SAMPLE_K)
        c = np.zeros((L, L), np.float32)
        np.add.at(c, (idx.ravel(), np.repeat(np.arange(L), SAMPLE_K)), 1.0)
        _cntT_cache = c
    return _cntT_cache


def _proj_body(x_ref, k_ref, v_ref, wq_ref, wk_ref, wv_ref,
               qo_ref, ko_ref, vo_ref):
    h = pl.program_id(1)
    dn = (((1,), (1,)), ((), ()))
    wq = wq_ref[pl.ds(h * DK, DK), :]
    wk = wk_ref[pl.ds(h * DK, DK), :]
    wv = wv_ref[pl.ds(h * DK, DK), :]
    qo_ref[...] = jax.lax.dot_general(x_ref[...], wq, dn,
                                      preferred_element_type=jnp.float32)[None]
    ko_ref[...] = jax.lax.dot_general(k_ref[...], wk, dn,
                                      preferred_element_type=jnp.float32)[None]
    vo_ref[...] = jax.lax.dot_general(v_ref[...], wv, dn,
                                      preferred_element_type=jnp.float32)[None]


def _project(q2, k2, v2, wq, wk, wv):
    row = pl.BlockSpec((LB, D), lambda i, h: (i, 0))
    wsp = pl.BlockSpec((D, D), lambda i, h: (0, 0))
    osp = pl.BlockSpec((1, LB, DK), lambda i, h: (h, i, 0))
    return pl.pallas_call(
        _proj_body,
        grid=(L // LB, H),
        in_specs=[row, row, row, wsp, wsp, wsp],
        out_specs=[osp, osp, osp],
        out_shape=[jax.ShapeDtypeStruct((H, L, DK), jnp.float32)] * 3,
    )(q2, k2, v2, wq, wk, wv)


def _m_body(qh_ref, kb_ref, cnt_ref, m_ref, mmax_sc, msum_sc):
    kb = pl.program_id(0)
    h = pl.program_id(1)
    st = jax.lax.dot_general(kb_ref[0], qh_ref[0], (((1,), (1,)), ((), ())),
                             preferred_element_type=jnp.float32, precision=jax.lax.Precision.HIGHEST)
    c = cnt_ref[...]
    neg = jnp.float32(-jnp.inf)
    bm = jnp.max(jnp.where(c > 0, st, neg), axis=0, keepdims=True)
    bs = jnp.sum(st * c, axis=0, keepdims=True)
    prev_max = jnp.where(kb == 0, jnp.full_like(bm, neg), mmax_sc[pl.ds(h, 1), :])
    prev_sum = jnp.where(kb == 0, jnp.zeros_like(bs), msum_sc[pl.ds(h, 1), :])
    new_max = jnp.maximum(prev_max, bm)
    new_sum = prev_sum + bs
    mmax_sc[pl.ds(h, 1), :] = new_max
    msum_sc[pl.ds(h, 1), :] = new_sum
    m_ref[...] = (new_max - new_sum * jnp.float32(1.0 / SAMPLE_K))[None]


def _sparsity_measure(qp, kp, cntT):
    return pl.pallas_call(
        _m_body,
        grid=(NKB, H),
        in_specs=[
            pl.BlockSpec((1, L, DK), lambda kb, h: (h, 0, 0)),
            pl.BlockSpec((1, KB, DK), lambda kb, h: (h, kb, 0)),
            pl.BlockSpec((KB, L), lambda kb, h: (kb, 0)),
        ],
        out_specs=pl.BlockSpec((1, 1, L), lambda kb, h: (h, 0, 0)),
        out_shape=jax.ShapeDtypeStruct((H, 1, L), jnp.float32),
        scratch_shapes=[
            pltpu.VMEM((H, L), jnp.float32),
            pltpu.VMEM((H, L), jnp.float32),
        ],
    )(qp, kp, cntT)


def _topk_body(m_ref, oh_ref, idx_ref):
    v = m_ref[:, 0, :]
    iota_l = jax.lax.broadcasted_iota(jnp.int32, (H, L), 1)
    iota_u = jax.lax.broadcasted_iota(jnp.int32, (H, UP), 1)
    idxs = jnp.zeros((H, UP), jnp.int32)
    for i in range(U):
        mx = jnp.max(v, axis=1, keepdims=True)
        eq = v == mx
        sel = jnp.min(jnp.where(eq, iota_l, L), axis=1, keepdims=True)
        idxs = jnp.where(iota_u == i, sel, idxs)
        v = jnp.where(iota_l == sel, jnp.float32(-jnp.inf), v)
    iota_l3 = jax.lax.broadcasted_iota(jnp.int32, (H, UP, L), 2)
    oh_ref[...] = (idxs[:, :, None] == iota_l3).astype(jnp.float32)
    idx_ref[...] = idxs[:, None, :]


def _topk(m):
    return pl.pallas_call(
        _topk_body,
        grid=(1,),
        in_specs=[pl.BlockSpec((H, 1, L), lambda i: (0, 0, 0))],
        out_specs=[
            pl.BlockSpec((H, UP, L), lambda i: (0, 0, 0)),
            pl.BlockSpec((H, 1, UP), lambda i: (0, 0, 0)),
        ],
        out_shape=[
            jax.ShapeDtypeStruct((H, UP, L), jnp.float32),
            jax.ShapeDtypeStruct((H, 1, UP), jnp.int32),
        ],
    )(m)


def _attn_body(oh_ref, qh_ref, kh_ref, vh_ref, wo_ref, c_ref, vm_ref):
    dn_nt = (((1,), (1,)), ((), ()))
    qr = jax.lax.dot_general(oh_ref[0], qh_ref[0], (((1,), (0,)), ((), ())),
                             preferred_element_type=jnp.float32, precision=jax.lax.Precision.HIGHEST)
    scores = jax.lax.dot_general(qr, kh_ref[0], dn_nt,
                                 preferred_element_type=jnp.float32, precision=jax.lax.Precision.HIGHEST)
    scores = scores * jnp.float32(SCALE)
    mr = jnp.max(scores, axis=1, keepdims=True)
    e = jnp.exp(scores - mr)
    attn = e / jnp.sum(e, axis=1, keepdims=True)
    upd = jax.lax.dot_general(attn, vh_ref[0], (((1,), (0,)), ((), ())),
                              preferred_element_type=jnp.float32, precision=jax.lax.Precision.HIGHEST)
    vmean = jnp.mean(vh_ref[0], axis=0, keepdims=True)
    delta = upd - vmean
    rmask = jax.lax.broadcasted_iota(jnp.int32, (UP, DK), 0) < U
    delta = jnp.where(rmask, delta, jnp.float32(0.0))
    ch = jax.lax.dot_general(delta, wo_ref[0], dn_nt,
                             preferred_element_type=jnp.float32, precision=jax.lax.Precision.HIGHEST)
    c_ref[...] = ch[None]
    vm_ref[...] = vmean[None]


def _sparse_attention(oh, qp, kp, vp, wo3):
    head_blk = pl.BlockSpec((1, L, DK), lambda h: (h, 0, 0))
    return pl.pallas_call(
        _attn_body,
        grid=(H,),
        in_specs=[
            pl.BlockSpec((1, UP, L), lambda h: (h, 0, 0)),
            head_blk, head_blk, head_blk,
            pl.BlockSpec((1, D, DK), lambda h: (h, 0, 0)),
        ],
        out_specs=[
            pl.BlockSpec((1, UP, D), lambda h: (h, 0, 0)),
            pl.BlockSpec((1, 1, DK), lambda h: (h, 0, 0)),
        ],
        out_shape=[
            jax.ShapeDtypeStruct((H, UP, D), jnp.float32),
            jax.ShapeDtypeStruct((H, 1, DK), jnp.float32),
        ],
    )(oh, qp, kp, vp, wo3)


def _out_body(idxrow_ref, vm_ref, wo_ref, bo_ref, c_ref, out_ref):
    i = pl.program_id(0)
    base = jax.lax.dot_general(vm_ref[...], wo_ref[...], (((1,), (1,)), ((), ())),
                               preferred_element_type=jnp.float32,
                               precision=jax.lax.Precision.HIGHEST)
    base = base + bo_ref[...]
    riota = jax.lax.broadcasted_iota(jnp.int32, (LB, H * UP), 0) + i * LB
    soh = (riota == idxrow_ref[...]).astype(jnp.float32)
    upd = jax.lax.dot_general(soh, c_ref[...], (((1,), (0,)), ((), ())),
                              preferred_element_type=jnp.float32)
    out_ref[...] = base + upd


def _assemble(idxrow, vm_flat, w_out, b_out2, c_flat):
    const_row = pl.BlockSpec((1, D), lambda i: (0, 0))
    return pl.pallas_call(
        _out_body,
        grid=(L // LB,),
        in_specs=[
            pl.BlockSpec((1, H * UP), lambda i: (0, 0)),
            const_row,
            pl.BlockSpec((D, D), lambda i: (0, 0)),
            const_row,
            pl.BlockSpec((H * UP, D), lambda i: (0, 0)),
        ],
        out_specs=pl.BlockSpec((LB, D), lambda i: (i, 0)),
        out_shape=jax.ShapeDtypeStruct((L, D), jnp.float32),
    )(idxrow, vm_flat, w_out, b_out2, c_flat)


def kernel(query, key, value, W_Q, W_K, W_V, W_out, b_out):
    q2, k2, v2 = query[0], key[0], value[0]
    cntT = jnp.asarray(_cntT_host())
    wo3 = W_out.reshape(D, H, DK).transpose(1, 0, 2)
    qp, kp, vp = _project(q2, k2, v2, W_Q, W_K, W_V)
    m = _sparsity_measure(qp, kp, cntT)
    oh, idx3 = _topk(m)
    c_all, vm3 = _sparse_attention(oh, qp, kp, vp, wo3)
    out = _assemble(
        idx3.reshape(1, H * UP),
        vm3.reshape(1, D),
        W_out,
        b_out.reshape(1, D),
        c_all.reshape(H * UP, D),
    )
    return out[None]


# speedup vs baseline: 1.1947x; 1.1947x over previous
import math
import functools

import numpy as np
import jax
import jax.numpy as jnp
from jax.experimental import pallas as pl
from jax.experimental.pallas import tpu as pltpu

L = 2048
D = 1024
H = 16
DK = 64
U = max(1, int(5 * math.log(L + 1)))
SAMPLE_K = max(1, int(5 * math.log(L + 1)))
KB = 512
NKB = L // KB
LB = 256
UP = 64
SCALE = 1.0 / math.sqrt(DK)

def _threefry_pair(k0, k1, x0, x1):
    rotations = [(13, 15, 26, 6), (17, 29, 16, 24)]
    ks = [np.uint32(k0), np.uint32(k1),
          np.uint32(k0) ^ np.uint32(k1) ^ np.uint32(0x1BD11BDA)]
    x = [x0.astype(np.uint32).copy(), x1.astype(np.uint32).copy()]
    with np.errstate(over="ignore"):
        x[0] += ks[0]
        x[1] += ks[1]
        for i in range(5):
            for r in rotations[i % 2]:
                x[0] += x[1]
                x[1] = (x[1] << np.uint32(r)) | (x[1] >> np.uint32(32 - r))
                x[1] ^= x[0]
            x[0] += ks[(i + 1) % 3]
            x[1] += ks[(i + 2) % 3] + np.uint32(i + 1)
    return x[0], x[1]


_cntT_cache = None


def _cntT_host():
    global _cntT_cache
    if _cntT_cache is None:
        b0, b1 = _threefry_pair(0, 42, np.zeros(2, np.uint32),
                                np.arange(2, dtype=np.uint32))
        n = L * SAMPLE_K
        c0, c1 = _threefry_pair(b0[1], b1[1], np.zeros(n, np.uint32),
                                np.arange(n, dtype=np.uint32))
        idx = ((c0 ^ c1) % np.uint32(L)).astype(np.int32).reshape(L, SAMPLE_K)
        c = np.zeros((L, L), np.float32)
        np.add.at(c, (idx.ravel(), np.repeat(np.arange(L), SAMPLE_K)), 1.0)
        _cntT_cache = c
    return _cntT_cache


def _proj_body(x_ref, k_ref, v_ref, wq_ref, wk_ref, wv_ref,
               qo_ref, ko_ref, vo_ref):
    dn = (((1,), (1,)), ((), ()))

    def proj(xr, wr, outr):
        y = jax.lax.dot_general(xr[...], wr[...], dn,
                                preferred_element_type=jnp.float32)
        outr[...] = y.reshape(LB, H, DK).transpose(1, 0, 2)

    proj(x_ref, wq_ref, qo_ref)
    proj(k_ref, wk_ref, ko_ref)
    proj(v_ref, wv_ref, vo_ref)


def _project(q2, k2, v2, wq, wk, wv):
    row = pl.BlockSpec((LB, D), lambda i: (i, 0))
    wsp = pl.BlockSpec((D, D), lambda i: (0, 0))
    osp = pl.BlockSpec((H, LB, DK), lambda i: (0, i, 0))
    return pl.pallas_call(
        _proj_body,
        grid=(L // LB,),
        in_specs=[row, row, row, wsp, wsp, wsp],
        out_specs=[osp, osp, osp],
        out_shape=[jax.ShapeDtypeStruct((H, L, DK), jnp.float32)] * 3,
    )(q2, k2, v2, wq, wk, wv)


def _m_body(qh_ref, kb_ref, cnt_ref, m_ref, mmax_sc, msum_sc):
    kb = pl.program_id(0)
    h = pl.program_id(1)
    st = jax.lax.dot_general(kb_ref[0], qh_ref[0], (((1,), (1,)), ((), ())),
                             preferred_element_type=jnp.float32, precision=jax.lax.Precision.HIGHEST)
    c = cnt_ref[...]
    neg = jnp.float32(-jnp.inf)
    bm = jnp.max(jnp.where(c > 0, st, neg), axis=0, keepdims=True)
    bs = jnp.sum(st * c, axis=0, keepdims=True)
    prev_max = jnp.where(kb == 0, jnp.full_like(bm, neg), mmax_sc[pl.ds(h, 1), :])
    prev_sum = jnp.where(kb == 0, jnp.zeros_like(bs), msum_sc[pl.ds(h, 1), :])
    new_max = jnp.maximum(prev_max, bm)
    new_sum = prev_sum + bs
    mmax_sc[pl.ds(h, 1), :] = new_max
    msum_sc[pl.ds(h, 1), :] = new_sum
    m_ref[...] = (new_max - new_sum * jnp.float32(1.0 / SAMPLE_K))[None]


def _sparsity_measure(qp, kp, cntT):
    return pl.pallas_call(
        _m_body,
        grid=(NKB, H),
        in_specs=[
            pl.BlockSpec((1, L, DK), lambda kb, h: (h, 0, 0)),
            pl.BlockSpec((1, KB, DK), lambda kb, h: (h, kb, 0)),
            pl.BlockSpec((KB, L), lambda kb, h: (kb, 0)),
        ],
        out_specs=pl.BlockSpec((1, 1, L), lambda kb, h: (h, 0, 0)),
        out_shape=jax.ShapeDtypeStruct((H, 1, L), jnp.float32),
        scratch_shapes=[
            pltpu.VMEM((H, L), jnp.float32),
            pltpu.VMEM((H, L), jnp.float32),
        ],
    )(qp, kp, cntT)


def _topk_body(m_ref, oh_ref, idx_ref):
    v = m_ref[:, 0, :]
    iota_l = jax.lax.broadcasted_iota(jnp.int32, (H, L), 1)
    iota_u = jax.lax.broadcasted_iota(jnp.int32, (H, UP), 1)
    idxs = jnp.zeros((H, UP), jnp.int32)
    for i in range(U):
        mx = jnp.max(v, axis=1, keepdims=True)
        eq = v == mx
        sel = jnp.min(jnp.where(eq, iota_l, L), axis=1, keepdims=True)
        idxs = jnp.where(iota_u == i, sel, idxs)
        v = jnp.where(iota_l == sel, jnp.float32(-jnp.inf), v)
    iota_l3 = jax.lax.broadcasted_iota(jnp.int32, (H, UP, L), 2)
    oh_ref[...] = (idxs[:, :, None] == iota_l3).astype(jnp.float32)
    idx_ref[...] = idxs[:, None, :]


def _topk(m):
    return pl.pallas_call(
        _topk_body,
        grid=(1,),
        in_specs=[pl.BlockSpec((H, 1, L), lambda i: (0, 0, 0))],
        out_specs=[
            pl.BlockSpec((H, UP, L), lambda i: (0, 0, 0)),
            pl.BlockSpec((H, 1, UP), lambda i: (0, 0, 0)),
        ],
        out_shape=[
            jax.ShapeDtypeStruct((H, UP, L), jnp.float32),
            jax.ShapeDtypeStruct((H, 1, UP), jnp.int32),
        ],
    )(m)


def _attn_body(oh_ref, qh_ref, kh_ref, vh_ref, wo_ref, c_ref, vm_ref):
    dn_nt = (((1,), (1,)), ((), ()))
    qr = jax.lax.dot_general(oh_ref[0], qh_ref[0], (((1,), (0,)), ((), ())),
                             preferred_element_type=jnp.float32, precision=jax.lax.Precision.HIGHEST)
    scores = jax.lax.dot_general(qr, kh_ref[0], dn_nt,
                                 preferred_element_type=jnp.float32, precision=jax.lax.Precision.HIGHEST)
    scores = scores * jnp.float32(SCALE)
    mr = jnp.max(scores, axis=1, keepdims=True)
    e = jnp.exp(scores - mr)
    attn = e / jnp.sum(e, axis=1, keepdims=True)
    upd = jax.lax.dot_general(attn, vh_ref[0], (((1,), (0,)), ((), ())),
                              preferred_element_type=jnp.float32, precision=jax.lax.Precision.HIGHEST)
    vmean = jnp.mean(vh_ref[0], axis=0, keepdims=True)
    delta = upd - vmean
    rmask = jax.lax.broadcasted_iota(jnp.int32, (UP, DK), 0) < U
    delta = jnp.where(rmask, delta, jnp.float32(0.0))
    ch = jax.lax.dot_general(delta, wo_ref[0], dn_nt,
                             preferred_element_type=jnp.float32, precision=jax.lax.Precision.HIGHEST)
    c_ref[...] = ch[None]
    vm_ref[...] = vmean[None]


def _sparse_attention(oh, qp, kp, vp, wo3):
    head_blk = pl.BlockSpec((1, L, DK), lambda h: (h, 0, 0))
    return pl.pallas_call(
        _attn_body,
        grid=(H,),
        in_specs=[
            pl.BlockSpec((1, UP, L), lambda h: (h, 0, 0)),
            head_blk, head_blk, head_blk,
            pl.BlockSpec((1, D, DK), lambda h: (h, 0, 0)),
        ],
        out_specs=[
            pl.BlockSpec((1, UP, D), lambda h: (h, 0, 0)),
            pl.BlockSpec((1, 1, DK), lambda h: (h, 0, 0)),
        ],
        out_shape=[
            jax.ShapeDtypeStruct((H, UP, D), jnp.float32),
            jax.ShapeDtypeStruct((H, 1, DK), jnp.float32),
        ],
    )(oh, qp, kp, vp, wo3)


def _out_body(s_ref, vm_ref, wo_ref, bo_ref, c_ref, out_ref):
    base = jax.lax.dot_general(vm_ref[...], wo_ref[...], (((1,), (1,)), ((), ())),
                               preferred_element_type=jnp.float32, precision=jax.lax.Precision.HIGHEST)
    base = base + bo_ref[...]
    out_ref[...] = jnp.broadcast_to(base, (L, D))

    def body(j, carry):
        h = j // U
        i = j - h * U
        r = s_ref[h * UP + i]
        row = c_ref[h, pl.ds(i, 1), :]
        out_ref[pl.ds(r, 1), :] = out_ref[pl.ds(r, 1), :] + row
        return carry

    jax.lax.fori_loop(0, H * U, body, 0)


def _assemble(idx_flat, vm_flat, w_out, b_out2, c_all):
    grid_spec = pltpu.PrefetchScalarGridSpec(
        num_scalar_prefetch=1,
        grid=(1,),
        in_specs=[
            pl.BlockSpec((1, D), lambda i, s: (0, 0)),
            pl.BlockSpec((D, D), lambda i, s: (0, 0)),
            pl.BlockSpec((1, D), lambda i, s: (0, 0)),
            pl.BlockSpec((H, UP, D), lambda i, s: (0, 0, 0)),
        ],
        out_specs=pl.BlockSpec((L, D), lambda i, s: (0, 0)),
    )
    return pl.pallas_call(
        _out_body,
        grid_spec=grid_spec,
        out_shape=jax.ShapeDtypeStruct((L, D), jnp.float32),
    )(idx_flat, vm_flat, w_out, b_out2, c_all)


def kernel(query, key, value, W_Q, W_K, W_V, W_out, b_out):
    q2, k2, v2 = query[0], key[0], value[0]
    cntT = jnp.asarray(_cntT_host())
    wo3 = W_out.reshape(D, H, DK).transpose(1, 0, 2)
    qp, kp, vp = _project(q2, k2, v2, W_Q, W_K, W_V)
    m = _sparsity_measure(qp, kp, cntT)
    oh, idx3 = _topk(m)
    c_all, vm3 = _sparse_attention(oh, qp, kp, vp, wo3)
    out = _assemble(
        idx3.reshape(H * UP),
        vm3.reshape(1, D),
        W_out,
        b_out.reshape(1, D),
        c_all,
    )
    return out[None]


# speedup vs baseline: 1.2472x; 1.0439x over previous
import math
import functools

import numpy as np
import jax
import jax.numpy as jnp
from jax.experimental import pallas as pl
from jax.experimental.pallas import tpu as pltpu

L = 2048
D = 1024
H = 16
DK = 64
U = max(1, int(5 * math.log(L + 1)))
SAMPLE_K = max(1, int(5 * math.log(L + 1)))
KB = 1024
NKB = L // KB
LB = 256
UP = 64
SCALE = 1.0 / math.sqrt(DK)

def _threefry_pair(k0, k1, x0, x1):
    rotations = [(13, 15, 26, 6), (17, 29, 16, 24)]
    ks = [np.uint32(k0), np.uint32(k1),
          np.uint32(k0) ^ np.uint32(k1) ^ np.uint32(0x1BD11BDA)]
    x = [x0.astype(np.uint32).copy(), x1.astype(np.uint32).copy()]
    with np.errstate(over="ignore"):
        x[0] += ks[0]
        x[1] += ks[1]
        for i in range(5):
            for r in rotations[i % 2]:
                x[0] += x[1]
                x[1] = (x[1] << np.uint32(r)) | (x[1] >> np.uint32(32 - r))
                x[1] ^= x[0]
            x[0] += ks[(i + 1) % 3]
            x[1] += ks[(i + 2) % 3] + np.uint32(i + 1)
    return x[0], x[1]


_cntT_cache = None


def _cntT_host():
    global _cntT_cache
    if _cntT_cache is None:
        b0, b1 = _threefry_pair(0, 42, np.zeros(2, np.uint32),
                                np.arange(2, dtype=np.uint32))
        n = L * SAMPLE_K
        c0, c1 = _threefry_pair(b0[1], b1[1], np.zeros(n, np.uint32),
                                np.arange(n, dtype=np.uint32))
        idx = ((c0 ^ c1) % np.uint32(L)).astype(np.int32).reshape(L, SAMPLE_K)
        c = np.zeros((L, L), np.float32)
        np.add.at(c, (idx.ravel(), np.repeat(np.arange(L), SAMPLE_K)), 1.0)
        _cntT_cache = c
    return _cntT_cache


def _proj_body(x_ref, k_ref, v_ref, wq_ref, wk_ref, wv_ref,
               qo_ref, ko_ref, vo_ref):
    dn = (((1,), (1,)), ((), ()))

    def proj(xr, wr, outr):
        y = jax.lax.dot_general(xr[...], wr[...], dn,
                                preferred_element_type=jnp.float32)
        outr[...] = y.reshape(LB, H, DK).transpose(1, 0, 2)

    proj(x_ref, wq_ref, qo_ref)
    proj(k_ref, wk_ref, ko_ref)
    proj(v_ref, wv_ref, vo_ref)


def _project(q2, k2, v2, wq, wk, wv):
    row = pl.BlockSpec((LB, D), lambda i: (i, 0))
    wsp = pl.BlockSpec((D, D), lambda i: (0, 0))
    osp = pl.BlockSpec((H, LB, DK), lambda i: (0, i, 0))
    return pl.pallas_call(
        _proj_body,
        grid=(L // LB,),
        in_specs=[row, row, row, wsp, wsp, wsp],
        out_specs=[osp, osp, osp],
        out_shape=[jax.ShapeDtypeStruct((H, L, DK), jnp.float32)] * 3,
    )(q2, k2, v2, wq, wk, wv)


def _m_body(qh_ref, kb_ref, cnt_ref, m_ref, mmax_sc, msum_sc):
    kb = pl.program_id(0)
    h = pl.program_id(1)
    st = jax.lax.dot_general(kb_ref[0], qh_ref[0], (((1,), (1,)), ((), ())),
                             preferred_element_type=jnp.float32, precision=jax.lax.Precision.HIGHEST)
    c = cnt_ref[...]
    neg = jnp.float32(-jnp.inf)
    bm = jnp.max(jnp.where(c > 0, st, neg), axis=0, keepdims=True)
    bs = jnp.sum(st * c, axis=0, keepdims=True)
    prev_max = jnp.where(kb == 0, jnp.full_like(bm, neg), mmax_sc[pl.ds(h, 1), :])
    prev_sum = jnp.where(kb == 0, jnp.zeros_like(bs), msum_sc[pl.ds(h, 1), :])
    new_max = jnp.maximum(prev_max, bm)
    new_sum = prev_sum + bs
    mmax_sc[pl.ds(h, 1), :] = new_max
    msum_sc[pl.ds(h, 1), :] = new_sum
    m_ref[...] = (new_max - new_sum * jnp.float32(1.0 / SAMPLE_K))[None]


def _sparsity_measure(qp, kp, cntT):
    return pl.pallas_call(
        _m_body,
        grid=(NKB, H),
        in_specs=[
            pl.BlockSpec((1, L, DK), lambda kb, h: (h, 0, 0)),
            pl.BlockSpec((1, KB, DK), lambda kb, h: (h, kb, 0)),
            pl.BlockSpec((KB, L), lambda kb, h: (kb, 0)),
        ],
        out_specs=pl.BlockSpec((1, 1, L), lambda kb, h: (h, 0, 0)),
        out_shape=jax.ShapeDtypeStruct((H, 1, L), jnp.float32),
        scratch_shapes=[
            pltpu.VMEM((H, L), jnp.float32),
            pltpu.VMEM((H, L), jnp.float32),
        ],
    )(qp, kp, cntT)


def _topk_body(m_ref, oh_ref, idx_ref):
    v = m_ref[:, 0, :]
    iota_l = jax.lax.broadcasted_iota(jnp.int32, (H, L), 1)
    iota_u = jax.lax.broadcasted_iota(jnp.int32, (H, UP), 1)
    idxs = jnp.zeros((H, UP), jnp.int32)
    for i in range(U):
        mx = jnp.max(v, axis=1, keepdims=True)
        eq = v == mx
        sel = jnp.min(jnp.where(eq, iota_l, L), axis=1, keepdims=True)
        idxs = jnp.where(iota_u == i, sel, idxs)
        v = jnp.where(iota_l == sel, jnp.float32(-jnp.inf), v)
    iota_l3 = jax.lax.broadcasted_iota(jnp.int32, (H, UP, L), 2)
    oh_ref[...] = (idxs[:, :, None] == iota_l3).astype(jnp.float32)
    idx_ref[...] = idxs[:, None, :]


def _topk(m):
    return pl.pallas_call(
        _topk_body,
        grid=(1,),
        in_specs=[pl.BlockSpec((H, 1, L), lambda i: (0, 0, 0))],
        out_specs=[
            pl.BlockSpec((H, UP, L), lambda i: (0, 0, 0)),
            pl.BlockSpec((H, 1, UP), lambda i: (0, 0, 0)),
        ],
        out_shape=[
            jax.ShapeDtypeStruct((H, UP, L), jnp.float32),
            jax.ShapeDtypeStruct((H, 1, UP), jnp.int32),
        ],
    )(m)


def _attn_body(oh_ref, qh_ref, kh_ref, vh_ref, wo_ref, c_ref, vm_ref):
    dn_nt = (((1,), (1,)), ((), ()))
    qr = jax.lax.dot_general(oh_ref[0], qh_ref[0], (((1,), (0,)), ((), ())),
                             preferred_element_type=jnp.float32, precision=jax.lax.Precision.HIGHEST)
    scores = jax.lax.dot_general(qr, kh_ref[0], dn_nt,
                                 preferred_element_type=jnp.float32, precision=jax.lax.Precision.HIGHEST)
    scores = scores * jnp.float32(SCALE)
    mr = jnp.max(scores, axis=1, keepdims=True)
    e = jnp.exp(scores - mr)
    attn = e / jnp.sum(e, axis=1, keepdims=True)
    upd = jax.lax.dot_general(attn, vh_ref[0], (((1,), (0,)), ((), ())),
                              preferred_element_type=jnp.float32, precision=jax.lax.Precision.HIGHEST)
    vmean = jnp.mean(vh_ref[0], axis=0, keepdims=True)
    delta = upd - vmean
    rmask = jax.lax.broadcasted_iota(jnp.int32, (UP, DK), 0) < U
    delta = jnp.where(rmask, delta, jnp.float32(0.0))
    ch = jax.lax.dot_general(delta, wo_ref[0], dn_nt,
                             preferred_element_type=jnp.float32, precision=jax.lax.Precision.HIGHEST)
    c_ref[...] = ch[None]
    vm_ref[...] = vmean[None]


def _sparse_attention(oh, qp, kp, vp, wo3):
    head_blk = pl.BlockSpec((1, L, DK), lambda h: (h, 0, 0))
    return pl.pallas_call(
        _attn_body,
        grid=(H,),
        in_specs=[
            pl.BlockSpec((1, UP, L), lambda h: (h, 0, 0)),
            head_blk, head_blk, head_blk,
            pl.BlockSpec((1, D, DK), lambda h: (h, 0, 0)),
        ],
        out_specs=[
            pl.BlockSpec((1, UP, D), lambda h: (h, 0, 0)),
            pl.BlockSpec((1, 1, DK), lambda h: (h, 0, 0)),
        ],
        out_shape=[
            jax.ShapeDtypeStruct((H, UP, D), jnp.float32),
            jax.ShapeDtypeStruct((H, 1, DK), jnp.float32),
        ],
    )(oh, qp, kp, vp, wo3)


def _out_body(s_ref, vm_ref, wo_ref, bo_ref, c_ref, out_ref):
    base = jax.lax.dot_general(vm_ref[...], wo_ref[...], (((1,), (1,)), ((), ())),
                               preferred_element_type=jnp.float32, precision=jax.lax.Precision.HIGHEST)
    base = base + bo_ref[...]
    out_ref[...] = jnp.broadcast_to(base, (L, D))

    def body(j, carry):
        h = j // U
        i = j - h * U
        r = s_ref[h * UP + i]
        row = c_ref[h, pl.ds(i, 1), :]
        out_ref[pl.ds(r, 1), :] = out_ref[pl.ds(r, 1), :] + row
        return carry

    jax.lax.fori_loop(0, H * U, body, 0)


def _assemble(idx_flat, vm_flat, w_out, b_out2, c_all):
    grid_spec = pltpu.PrefetchScalarGridSpec(
        num_scalar_prefetch=1,
        grid=(1,),
        in_specs=[
            pl.BlockSpec((1, D), lambda i, s: (0, 0)),
            pl.BlockSpec((D, D), lambda i, s: (0, 0)),
            pl.BlockSpec((1, D), lambda i, s: (0, 0)),
            pl.BlockSpec((H, UP, D), lambda i, s: (0, 0, 0)),
        ],
        out_specs=pl.BlockSpec((L, D), lambda i, s: (0, 0)),
    )
    return pl.pallas_call(
        _out_body,
        grid_spec=grid_spec,
        out_shape=jax.ShapeDtypeStruct((L, D), jnp.float32),
    )(idx_flat, vm_flat, w_out, b_out2, c_all)


def kernel(query, key, value, W_Q, W_K, W_V, W_out, b_out):
    q2, k2, v2 = query[0], key[0], value[0]
    cntT = jnp.asarray(_cntT_host())
    wo3 = W_out.reshape(D, H, DK).transpose(1, 0, 2)
    qp, kp, vp = _project(q2, k2, v2, W_Q, W_K, W_V)
    m = _sparsity_measure(qp, kp, cntT)
    oh, idx3 = _topk(m)
    c_all, vm3 = _sparse_attention(oh, qp, kp, vp, wo3)
    out = _assemble(
        idx3.reshape(H * UP),
        vm3.reshape(1, D),
        W_out,
        b_out.reshape(1, D),
        c_all,
    )
    return out[None]
